# baseline (speedup 1.0000x reference)
import functools

import jax
import jax.numpy as jnp
from jax import lax
from jax.experimental import pallas as pl
from jax.experimental.pallas import tpu as pltpu
from jax.experimental.pallas import tpu_sc as plsc

N = 10000
E = 320000
D = 128

NC = 2
NS = 16
NW = NC * NS

CHUNK = 128
N_CHUNKS = E // CHUNK
BASE_K = N_CHUNKS // NW
EXTRA = N_CHUNKS - BASE_K * NW
BASE_E = BASE_K * CHUNK
NBUF = 2
TRIPS = (BASE_K + 2) // NBUF

ROWS_MAIN = 624
REM_BASE = NS * ROWS_MAIN
REM_ROWS = N - REM_BASE
ZROWS = 16


def _sc_kernel(x_hbm, src_hbm, dst_hbm, out_hbm,
               idx_src, idx_dst, rows0, rows1, zbuf, acc,
               bsem, zsem, isem0, isem1, gsem0, gsem1, ssem0, ssem1):
    c = lax.axis_index("c")
    s = lax.axis_index("s")
    wid = c * NS + s

    nk = jnp.where(wid < EXTRA, BASE_K + 1, BASE_K)
    base = wid * BASE_K + jnp.minimum(wid, EXTRA)

    rows = (rows0, rows1)
    isem = (isem0, isem1)
    gsem = (gsem0, gsem1)
    ssem = (ssem0, ssem1)

    def fire_idx(k, b):
        pltpu.async_copy(dst_hbm.at[pl.ds(base + k, 1)],
                         idx_dst.at[pl.ds(b, 1)], isem[b])

    def fire_gather(k, b):
        pltpu.async_copy(x_hbm.at[idx_src.at[pl.ds(k * CHUNK, CHUNK)]],
                         rows[b], gsem[b])

    bp = pltpu.async_copy(src_hbm.at[pl.ds(base * CHUNK, BASE_E)],
                          idx_src.at[pl.ds(0, BASE_E)], bsem)

    @pl.when(wid < EXTRA)
    def _():
        pltpu.async_copy(src_hbm.at[pl.ds(base * CHUNK + BASE_E, CHUNK)],
                         idx_src.at[pl.ds(BASE_E, CHUNK)], bsem)

    for b in range(NBUF):
        fire_idx(b, b)

    def zero_row(i, _):
        for j in range(D // 16):
            zbuf[i, pl.ds(j * 16, 16)] = jnp.zeros((16,), jnp.float32)
        return 0
    lax.fori_loop(0, ZROWS, zero_row, 0)

    def zero_acc(i, _):
        pltpu.async_copy(zbuf,
                         acc.at[pl.ds(s * ROWS_MAIN + i * ZROWS, ZROWS)],
                         zsem)
        return 0
    lax.fori_loop(0, ROWS_MAIN // ZROWS, zero_acc, 0)

    @pl.when(s == 0)
    def _():
        pltpu.async_copy(zbuf.at[pl.ds(0, REM_ROWS)],
                         acc.at[pl.ds(REM_BASE, REM_ROWS)], zsem)

    bp.wait()

    @pl.when(wid < EXTRA)
    def _():
        pltpu.make_async_copy(src_hbm.at[pl.ds(0, CHUNK)],
                              idx_src.at[pl.ds(BASE_E, CHUNK)],
                              bsem).wait()

    for b in range(NBUF):
        fire_gather(b, b)

    def drain_zero(i, _):
        pltpu.make_async_copy(x_hbm.at[pl.ds(0, ZROWS)],
                              acc.at[pl.ds(s * ROWS_MAIN + i * ZROWS, ZROWS)],
                              zsem).wait()
        return 0
    lax.fori_loop(0, ROWS_MAIN // ZROWS, drain_zero, 0)

    @pl.when(s == 0)
    def _():
        pltpu.make_async_copy(x_hbm.at[pl.ds(0, REM_ROWS)],
                              acc.at[pl.ds(REM_BASE, REM_ROWS)],
                              zsem).wait()

    plsc.subcore_barrier()

    def scatter(k, b):
        pltpu.async_copy(rows[b], acc.at[idx_dst.at[b]], ssem[b], add=True)

    def wait_slot(b):
        pltpu.make_async_copy(x_hbm.at[pl.ds(0, CHUNK)], rows[b],
                              gsem[b]).wait()
        pltpu.make_async_copy(dst_hbm.at[pl.ds(0, 1)],
                              idx_dst.at[pl.ds(b, 1)], isem[b]).wait()

    def wait_scat(b):
        pltpu.make_async_copy(x_hbm.at[pl.ds(0, CHUNK)], rows[b],
                              ssem[b]).wait()

    def body(g, _):
        k0 = g * NBUF
        for b in range(NBUF):
            k = k0 + b

            @pl.when(k < nk)
            def _():
                wait_slot(b)
                scatter(k, b)

                @pl.when(k + NBUF < nk)
                def _():
                    wait_scat(b)
                    fire_idx(k + NBUF, b)
                    fire_gather(k + NBUF, b)
        return 0
    lax.fori_loop(0, TRIPS, body, 0)

    for b in range(NBUF):
        wait_scat(b)

    plsc.subcore_barrier()

    pltpu.sync_copy(acc.at[pl.ds(s * ROWS_MAIN, ROWS_MAIN)],
                    out_hbm.at[c, pl.ds(s * ROWS_MAIN, ROWS_MAIN)])

    @pl.when(s == 0)
    def _():
        pltpu.sync_copy(acc.at[pl.ds(REM_BASE, REM_ROWS)],
                        out_hbm.at[c, pl.ds(REM_BASE, REM_ROWS)])


@jax.jit
def _sc_scatter(x, src, dst):
    mesh = plsc.VectorSubcoreMesh(core_axis_name="c", subcore_axis_name="s")
    return pl.kernel(
        _sc_kernel,
        out_type=jax.ShapeDtypeStruct((NC, N, D), jnp.float32),
        mesh=mesh,
        scratch_types=[
            pltpu.VMEM((BASE_E + CHUNK,), jnp.int32),
            pltpu.VMEM((NBUF, CHUNK), jnp.int32),
            pltpu.VMEM((CHUNK, D), jnp.float32),
            pltpu.VMEM((CHUNK, D), jnp.float32),
            pltpu.VMEM((ZROWS, D), jnp.float32),
            pltpu.VMEM_SHARED((N, D), jnp.float32),
            pltpu.SemaphoreType.DMA,
            pltpu.SemaphoreType.DMA,
            pltpu.SemaphoreType.DMA,
            pltpu.SemaphoreType.DMA,
            pltpu.SemaphoreType.DMA,
            pltpu.SemaphoreType.DMA,
            pltpu.SemaphoreType.DMA,
            pltpu.SemaphoreType.DMA,
        ],
    )(x, src, dst)


def _add_body(p_ref, o_ref):
    o_ref[...] = p_ref[0] + p_ref[1]


BLK = 1000


@jax.jit
def _combine(partials):
    return pl.pallas_call(
        _add_body,
        grid=(N // BLK,),
        in_specs=[pl.BlockSpec((NC, BLK, D), lambda i: (0, i, 0))],
        out_specs=pl.BlockSpec((BLK, D), lambda i: (i, 0)),
        out_shape=jax.ShapeDtypeStruct((N, D), jnp.float32),
    )(partials)


def kernel(x, edge_index):
    src = edge_index[0]
    dst = edge_index[1].reshape(N_CHUNKS, CHUNK)
    partials = _sc_scatter(x, src, dst)
    return _combine(partials)


# speedup vs baseline: 1.0467x; 1.0467x over previous
import functools

import jax
import jax.numpy as jnp
from jax import lax
from jax.experimental import pallas as pl
from jax.experimental.pallas import tpu as pltpu
from jax.experimental.pallas import tpu_sc as plsc

N = 10000
E = 320000
D = 128

NC = 2
NS = 16
NW = NC * NS

E_PER_TILE = E // NW
CHUNK = 80
N_CHUNKS = E_PER_TILE // CHUNK
NBUF = 3

N_MAIN = ((N_CHUNKS - NBUF) // NBUF) * NBUF

ROWS_MAIN = 624
REM_BASE = NS * ROWS_MAIN
REM_ROWS = N - REM_BASE
ZROWS = 16


def _sc_kernel(x_hbm, src_hbm, dst_hbm, out_hbm,
               idx_src, idx_dst, rows0, rows1, rows2, zbuf, acc,
               bsem, isem0, isem1, isem2, gsem0, gsem1, gsem2,
               ssem0, ssem1, ssem2):
    c = lax.axis_index("c")
    s = lax.axis_index("s")
    wid = c * NS + s

    rows = (rows0, rows1, rows2)
    isem = (isem0, isem1, isem2)
    gsem = (gsem0, gsem1, gsem2)
    ssem = (ssem0, ssem1, ssem2)

    def fire_idx(k, b):
        pltpu.async_copy(dst_hbm.at[wid, pl.ds(k, 1)],
                         idx_dst.at[pl.ds(b, 1)], isem[b])

    def fire_gather(k, b):
        pltpu.async_copy(x_hbm.at[idx_src.at[pl.ds(k * CHUNK, CHUNK)]],
                         rows[b], gsem[b])

    bp = pltpu.async_copy(src_hbm.at[pl.ds(wid * E_PER_TILE, E_PER_TILE)],
                          idx_src, bsem)
    for b in range(NBUF):
        fire_idx(b, b)

    def zero_row(i, _):
        for j in range(D // 16):
            zbuf[i, pl.ds(j * 16, 16)] = jnp.zeros((16,), jnp.float32)
        return 0
    lax.fori_loop(0, ZROWS, zero_row, 0)

    def zero_acc(i, _):
        pltpu.sync_copy(zbuf, acc.at[pl.ds(s * ROWS_MAIN + i * ZROWS, ZROWS)])
        return 0
    lax.fori_loop(0, ROWS_MAIN // ZROWS, zero_acc, 0)

    @pl.when(s == 0)
    def _():
        pltpu.sync_copy(zbuf.at[pl.ds(0, REM_ROWS)],
                        acc.at[pl.ds(REM_BASE, REM_ROWS)])

    bp.wait()
    for b in range(NBUF):
        fire_gather(b, b)

    plsc.subcore_barrier()

    def scatter(k, b):
        pltpu.async_copy(rows[b], acc.at[idx_dst.at[b]], ssem[b], add=True)

    def wait_slot(b):
        pltpu.make_async_copy(x_hbm.at[pl.ds(0, CHUNK)], rows[b],
                              gsem[b]).wait()
        pltpu.make_async_copy(dst_hbm.at[wid, pl.ds(0, 1)],
                              idx_dst.at[pl.ds(b, 1)], isem[b]).wait()

    def wait_scat(b):
        pltpu.make_async_copy(x_hbm.at[pl.ds(0, CHUNK)], rows[b],
                              ssem[b]).wait()

    def body(g, _):
        k0 = g * NBUF
        for b in range(NBUF):
            k = k0 + b
            wait_slot(b)
            scatter(k, b)
            wait_scat(b)
            fire_idx(k + NBUF, b)
            fire_gather(k + NBUF, b)
        return 0
    lax.fori_loop(0, N_MAIN // NBUF, body, 0)

    for k in range(N_MAIN, N_CHUNKS):
        b = k % NBUF
        wait_slot(b)
        scatter(k, b)
        if k + NBUF < N_CHUNKS:
            wait_scat(b)
            fire_idx(k + NBUF, b)
            fire_gather(k + NBUF, b)

    for b in range(NBUF):
        wait_scat(b)

    plsc.subcore_barrier()

    pltpu.sync_copy(acc.at[pl.ds(s * ROWS_MAIN, ROWS_MAIN)],
                    out_hbm.at[c, pl.ds(s * ROWS_MAIN, ROWS_MAIN)])

    @pl.when(s == 0)
    def _():
        pltpu.sync_copy(acc.at[pl.ds(REM_BASE, REM_ROWS)],
                        out_hbm.at[c, pl.ds(REM_BASE, REM_ROWS)])


@jax.jit
def _sc_scatter(x, src, dst):
    mesh = plsc.VectorSubcoreMesh(core_axis_name="c", subcore_axis_name="s")
    return pl.kernel(
        _sc_kernel,
        out_type=jax.ShapeDtypeStruct((NC, N, D), jnp.float32),
        mesh=mesh,
        scratch_types=[
            pltpu.VMEM((E_PER_TILE,), jnp.int32),
            pltpu.VMEM((NBUF, CHUNK), jnp.int32),
            pltpu.VMEM((CHUNK, D), jnp.float32),
            pltpu.VMEM((CHUNK, D), jnp.float32),
            pltpu.VMEM((CHUNK, D), jnp.float32),
            pltpu.VMEM((ZROWS, D), jnp.float32),
            pltpu.VMEM_SHARED((N, D), jnp.float32),
            pltpu.SemaphoreType.DMA,
            pltpu.SemaphoreType.DMA,
            pltpu.SemaphoreType.DMA,
            pltpu.SemaphoreType.DMA,
            pltpu.SemaphoreType.DMA,
            pltpu.SemaphoreType.DMA,
            pltpu.SemaphoreType.DMA,
            pltpu.SemaphoreType.DMA,
            pltpu.SemaphoreType.DMA,
            pltpu.SemaphoreType.DMA,
        ],
    )(x, src, dst)


def _add_body(p_ref, o_ref):
    o_ref[...] = p_ref[0] + p_ref[1]


BLK = 2000


@jax.jit
def _combine(partials):
    return pl.pallas_call(
        _add_body,
        grid=(N // BLK,),
        in_specs=[pl.BlockSpec((NC, BLK, D), lambda i: (0, i, 0))],
        out_specs=pl.BlockSpec((BLK, D), lambda i: (i, 0)),
        out_shape=jax.ShapeDtypeStruct((N, D), jnp.float32),
    )(partials)


def kernel(x, edge_index):
    src = edge_index[0]
    dst = edge_index[1].reshape(NW, N_CHUNKS, CHUNK)
    partials = _sc_scatter(x, src, dst)
    return _combine(partials)


# speedup vs baseline: 1.0555x; 1.0084x over previous
import functools

import jax
import jax.numpy as jnp
from jax import lax
from jax.experimental import pallas as pl
from jax.experimental.pallas import tpu as pltpu
from jax.experimental.pallas import tpu_sc as plsc

N = 10000
E = 320000
D = 128

NC = 2
NS = 16
NW = NC * NS

E_PER_TILE = E // NW
CHUNK = 80
N_CHUNKS = E_PER_TILE // CHUNK
NBUF = 3

N_MAIN = ((N_CHUNKS - NBUF) // NBUF) * NBUF

ROWS_MAIN = 624
REM_BASE = NS * ROWS_MAIN
REM_ROWS = N - REM_BASE
ZROWS = 48


def _sc_kernel(x_hbm, src_hbm, dst_hbm, out_hbm,
               idx_src, idx_dst, rows0, rows1, rows2, zbuf, acc,
               bsem, isem0, isem1, isem2, gsem0, gsem1, gsem2,
               ssem0, ssem1, ssem2):
    c = lax.axis_index("c")
    s = lax.axis_index("s")
    wid = c * NS + s

    rows = (rows0, rows1, rows2)
    isem = (isem0, isem1, isem2)
    gsem = (gsem0, gsem1, gsem2)
    ssem = (ssem0, ssem1, ssem2)

    def fire_idx(k, b):
        pltpu.async_copy(dst_hbm.at[wid, pl.ds(k, 1)],
                         idx_dst.at[pl.ds(b, 1)], isem[b])

    def fire_gather(k, b):
        pltpu.async_copy(x_hbm.at[idx_src.at[pl.ds(k * CHUNK, CHUNK)]],
                         rows[b], gsem[b])

    bp = pltpu.async_copy(src_hbm.at[pl.ds(wid * E_PER_TILE, E_PER_TILE)],
                          idx_src, bsem)
    for b in range(NBUF):
        fire_idx(b, b)

    def zero_row(i, _):
        for j in range(D // 16):
            zbuf[i, pl.ds(j * 16, 16)] = jnp.zeros((16,), jnp.float32)
        return 0
    lax.fori_loop(0, ZROWS, zero_row, 0)

    def zero_acc(i, _):
        pltpu.sync_copy(zbuf, acc.at[pl.ds(s * ROWS_MAIN + i * ZROWS, ZROWS)])
        return 0
    lax.fori_loop(0, ROWS_MAIN // ZROWS, zero_acc, 0)

    @pl.when(s == 0)
    def _():
        pltpu.sync_copy(zbuf.at[pl.ds(0, REM_ROWS)],
                        acc.at[pl.ds(REM_BASE, REM_ROWS)])

    bp.wait()
    for b in range(NBUF):
        fire_gather(b, b)

    plsc.subcore_barrier()

    def scatter(k, b):
        pltpu.async_copy(rows[b], acc.at[idx_dst.at[b]], ssem[b], add=True)

    def wait_slot(b):
        pltpu.make_async_copy(x_hbm.at[pl.ds(0, CHUNK)], rows[b],
                              gsem[b]).wait()
        pltpu.make_async_copy(dst_hbm.at[wid, pl.ds(0, 1)],
                              idx_dst.at[pl.ds(b, 1)], isem[b]).wait()

    def wait_scat(b):
        pltpu.make_async_copy(x_hbm.at[pl.ds(0, CHUNK)], rows[b],
                              ssem[b]).wait()

    def body(g, _):
        k0 = g * NBUF
        for b in range(NBUF):
            k = k0 + b
            wait_slot(b)
            scatter(k, b)
            wait_scat(b)
            fire_idx(k + NBUF, b)
            fire_gather(k + NBUF, b)
        return 0
    lax.fori_loop(0, N_MAIN // NBUF, body, 0)

    for k in range(N_MAIN, N_CHUNKS):
        b = k % NBUF
        wait_slot(b)
        scatter(k, b)
        if k + NBUF < N_CHUNKS:
            wait_scat(b)
            fire_idx(k + NBUF, b)
            fire_gather(k + NBUF, b)

    for b in range(NBUF):
        wait_scat(b)

    plsc.subcore_barrier()

    pltpu.sync_copy(acc.at[pl.ds(s * ROWS_MAIN, ROWS_MAIN)],
                    out_hbm.at[c, pl.ds(s * ROWS_MAIN, ROWS_MAIN)])

    @pl.when(s == 0)
    def _():
        pltpu.sync_copy(acc.at[pl.ds(REM_BASE, REM_ROWS)],
                        out_hbm.at[c, pl.ds(REM_BASE, REM_ROWS)])


@jax.jit
def _sc_scatter(x, src, dst):
    mesh = plsc.VectorSubcoreMesh(core_axis_name="c", subcore_axis_name="s")
    return pl.kernel(
        _sc_kernel,
        out_type=jax.ShapeDtypeStruct((NC, N, D), jnp.float32),
        mesh=mesh,
        scratch_types=[
            pltpu.VMEM((E_PER_TILE,), jnp.int32),
            pltpu.VMEM((NBUF, CHUNK), jnp.int32),
            pltpu.VMEM((CHUNK, D), jnp.float32),
            pltpu.VMEM((CHUNK, D), jnp.float32),
            pltpu.VMEM((CHUNK, D), jnp.float32),
            pltpu.VMEM((ZROWS, D), jnp.float32),
            pltpu.VMEM_SHARED((N, D), jnp.float32),
            pltpu.SemaphoreType.DMA,
            pltpu.SemaphoreType.DMA,
            pltpu.SemaphoreType.DMA,
            pltpu.SemaphoreType.DMA,
            pltpu.SemaphoreType.DMA,
            pltpu.SemaphoreType.DMA,
            pltpu.SemaphoreType.DMA,
            pltpu.SemaphoreType.DMA,
            pltpu.SemaphoreType.DMA,
            pltpu.SemaphoreType.DMA,
        ],
    )(x, src, dst)


def _add_body(p_ref, o_ref):
    o_ref[...] = p_ref[0] + p_ref[1]


BLK = 2000


@jax.jit
def _combine(partials):
    return pl.pallas_call(
        _add_body,
        grid=(N // BLK,),
        in_specs=[pl.BlockSpec((NC, BLK, D), lambda i: (0, i, 0))],
        out_specs=pl.BlockSpec((BLK, D), lambda i: (i, 0)),
        out_shape=jax.ShapeDtypeStruct((N, D), jnp.float32),
    )(partials)


def kernel(x, edge_index):
    src = edge_index[0]
    dst = edge_index[1].reshape(NW, N_CHUNKS, CHUNK)
    partials = _sc_scatter(x, src, dst)
    return _combine(partials)
